# baseline (speedup 1.0000x reference)
import jax
import jax.numpy as jnp
from jax import lax
from jax.experimental import pallas as pl
from jax.experimental.pallas import tpu as pltpu
from jax.experimental.pallas import tpu_sc as plsc

BATCH = 4096
EMB = 32
_NC = 2
_NS = 16
_NW = _NC * _NS
_BPW = BATCH // _NW


def _sc_gather_body(uids_hbm, iids_hbm, utab_hbm, itab_hbm, u_out, i_out,
                    uidx_v, iidx_v, urows_v, irows_v, usem, isem):
    wid = lax.axis_index("s") * _NC + lax.axis_index("c")
    base = wid * _BPW
    pltpu.sync_copy(uids_hbm.at[pl.ds(base, _BPW)], uidx_v)
    pltpu.sync_copy(iids_hbm.at[pl.ds(base, _BPW)], iidx_v)
    ucp = pltpu.async_copy(utab_hbm.at[uidx_v], urows_v, usem)
    icp = pltpu.async_copy(itab_hbm.at[iidx_v], irows_v, isem)
    ucp.wait()
    icp.wait()
    pltpu.sync_copy(urows_v, u_out.at[pl.ds(base, _BPW)])
    pltpu.sync_copy(irows_v, i_out.at[pl.ds(base, _BPW)])


def _sc_gather(user_ids, item_ids, user_table, item_table):
    mesh = plsc.VectorSubcoreMesh(core_axis_name="c", subcore_axis_name="s")
    kfn = pl.kernel(
        _sc_gather_body,
        mesh=mesh,
        out_type=[
            jax.ShapeDtypeStruct((BATCH, EMB), jnp.float32),
            jax.ShapeDtypeStruct((BATCH, EMB), jnp.float32),
        ],
        scratch_types=[
            pltpu.VMEM((_BPW,), jnp.int32),
            pltpu.VMEM((_BPW,), jnp.int32),
            pltpu.VMEM((_BPW, EMB), jnp.float32),
            pltpu.VMEM((_BPW, EMB), jnp.float32),
            pltpu.SemaphoreType.DMA,
            pltpu.SemaphoreType.DMA,
        ],
        compiler_params=pltpu.CompilerParams(use_tc_tiling_on_sc=False),
    )
    return kfn(user_ids, item_ids, user_table, item_table)


def _tc_dense_body(u_ref, i_ref, w1u_ref, w1i_ref, w1p_ref, b1_ref,
                   w2_ref, b2_ref, pred_ref, score_ref):
    u = u_ref[...]
    i = i_ref[...]
    s = jnp.sum(u, axis=0, keepdims=True)
    pred_ref[...] = jnp.sum(i * s, axis=1, keepdims=True)
    ui = u * i
    h = (jnp.dot(u, w1u_ref[...], preferred_element_type=jnp.float32)
         + jnp.dot(i, w1i_ref[...], preferred_element_type=jnp.float32)
         + jnp.dot(ui, w1p_ref[...], preferred_element_type=jnp.float32)
         + b1_ref[...])
    h = jnp.maximum(h, 0.0)
    score_ref[...] = (jnp.sum(h * w2_ref[...], axis=1, keepdims=True)
                      + b2_ref[...])


def _tc_dense(u, i, W1, b1, W2, b2):
    w1t = W1.T
    return pl.pallas_call(
        _tc_dense_body,
        out_shape=[
            jax.ShapeDtypeStruct((BATCH, 1), jnp.float32),
            jax.ShapeDtypeStruct((BATCH, 1), jnp.float32),
        ],
    )(u, i, w1t[:EMB], w1t[EMB:2 * EMB], w1t[2 * EMB:],
      b1.reshape(1, 64), W2.reshape(1, 64), b2.reshape(1, 1))


def kernel(user_ids, item_ids, user_table, item_table, W1, b1, W2, b2):
    uids = user_ids.astype(jnp.int32)
    iids = item_ids.astype(jnp.int32)
    u, i = _sc_gather(uids, iids, user_table, item_table)
    pred, score = _tc_dense(u, i, W1, b1, W2, b2)
    return (pred[:, 0], score[:, 0])


# speedup vs baseline: 10.1405x; 10.1405x over previous
import jax
import jax.numpy as jnp
from jax import lax
from jax.experimental import pallas as pl
from jax.experimental.pallas import tpu as pltpu
from jax.experimental.pallas import tpu_sc as plsc

BATCH = 4096
EMB = 32
_NC = 2
_NS = 16
_NW = _NC * _NS
_BPW = BATCH // _NW
_CHUNK = 16


def _gather_one(tab_t, idx_v, ring, obuf, sem):
    iota = lax.iota(jnp.int32, 16)

    def chunk(c, carry):
        vec = idx_v[pl.ds(c * _CHUNK, _CHUNK)]
        for l in range(_CHUNK):
            uid = vec[l]
            col = pl.multiple_of((uid // 128) * 128, 128)
            pltpu.async_copy(tab_t.at[:, pl.ds(col, 128)], ring.at[l], sem)
        for l in range(_CHUNK):
            pltpu.make_async_copy(tab_t.at[:, pl.ds(0, 128)], ring.at[l],
                                  sem).wait()
        for l in range(_CHUNK):
            uid = vec[l]
            lane = jnp.full((16,), uid % 128, jnp.int32)
            jcol = jnp.full((16,), c * _CHUNK + l, jnp.int32)
            lo = plsc.load_gather(ring.at[l], [iota, lane])
            hi = plsc.load_gather(ring.at[l], [iota + 16, lane])
            plsc.store_scatter(obuf, [iota, jcol], lo)
            plsc.store_scatter(obuf, [iota + 16, jcol], hi)
        return carry

    lax.fori_loop(0, _BPW // _CHUNK, chunk, 0)


def _sc_gather_body(uids_hbm, iids_hbm, utab_t, itab_t, ut_out, it_out,
                    uidx_v, iidx_v, ring, ubuf, ibuf, sem):
    wid = lax.axis_index("s") * _NC + lax.axis_index("c")
    base = wid * _BPW
    pltpu.sync_copy(uids_hbm.at[pl.ds(base, _BPW)], uidx_v)
    pltpu.sync_copy(iids_hbm.at[pl.ds(base, _BPW)], iidx_v)
    _gather_one(utab_t, uidx_v, ring, ubuf, sem)
    _gather_one(itab_t, iidx_v, ring, ibuf, sem)
    pltpu.sync_copy(ubuf, ut_out.at[:, pl.ds(base, _BPW)])
    pltpu.sync_copy(ibuf, it_out.at[:, pl.ds(base, _BPW)])


def _sc_gather(user_ids, item_ids, utab_t, itab_t):
    mesh = plsc.VectorSubcoreMesh(core_axis_name="c", subcore_axis_name="s")
    kfn = pl.kernel(
        _sc_gather_body,
        mesh=mesh,
        out_type=[
            jax.ShapeDtypeStruct((EMB, BATCH), jnp.float32),
            jax.ShapeDtypeStruct((EMB, BATCH), jnp.float32),
        ],
        scratch_types=[
            pltpu.VMEM((_BPW,), jnp.int32),
            pltpu.VMEM((_BPW,), jnp.int32),
            pltpu.VMEM((_CHUNK, EMB, 128), jnp.float32),
            pltpu.VMEM((EMB, _BPW), jnp.float32),
            pltpu.VMEM((EMB, _BPW), jnp.float32),
            pltpu.SemaphoreType.DMA,
        ],
        compiler_params=pltpu.CompilerParams(use_tc_tiling_on_sc=True,
                                             needs_layout_passes=False),
    )
    return kfn(user_ids, item_ids, utab_t, itab_t)


def _tc_dense_body(ut_ref, it_ref, w1u_ref, w1i_ref, w1p_ref, b1_ref,
                   w2_ref, b2_ref, pred_ref, score_ref):
    ut = ut_ref[...]
    it = it_ref[...]
    s = jnp.sum(ut, axis=1, keepdims=True)
    pred_ref[...] = jnp.sum(it * s, axis=0, keepdims=True)
    uit = ut * it
    h = (jnp.dot(w1u_ref[...], ut, preferred_element_type=jnp.float32)
         + jnp.dot(w1i_ref[...], it, preferred_element_type=jnp.float32)
         + jnp.dot(w1p_ref[...], uit, preferred_element_type=jnp.float32)
         + b1_ref[...])
    h = jnp.maximum(h, 0.0)
    score_ref[...] = (jnp.sum(h * w2_ref[...], axis=0, keepdims=True)
                      + b2_ref[...])


def _tc_dense(ut, it, W1, b1, W2, b2):
    return pl.pallas_call(
        _tc_dense_body,
        out_shape=[
            jax.ShapeDtypeStruct((1, BATCH), jnp.float32),
            jax.ShapeDtypeStruct((1, BATCH), jnp.float32),
        ],
    )(ut, it, W1[:, :EMB], W1[:, EMB:2 * EMB], W1[:, 2 * EMB:],
      b1.reshape(64, 1), W2.reshape(64, 1), b2.reshape(1, 1))


def kernel(user_ids, item_ids, user_table, item_table, W1, b1, W2, b2):
    uids = user_ids.astype(jnp.int32)
    iids = item_ids.astype(jnp.int32)
    ut, it = _sc_gather(uids, iids, user_table.T, item_table.T)
    pred, score = _tc_dense(ut, it, W1, b1, W2, b2)
    return (pred[0], score[0])


# speedup vs baseline: 12.1480x; 1.1980x over previous
import jax
import jax.numpy as jnp
from jax import lax
from jax.experimental import pallas as pl
from jax.experimental.pallas import tpu as pltpu
from jax.experimental.pallas import tpu_sc as plsc

BATCH = 4096
EMB = 32
_NC = 2
_NS = 16
_NW = _NC * _NS
_BPW = BATCH // _NW
_CHUNK = 8
_IDXPAD = _BPW + 32


def _gather_one(tab_t, idx_v, ring, obuf, sem):
    iota = lax.iota(jnp.int32, 16)
    nchunk = _BPW // _CHUNK

    def fire(k, buf):
        vec = idx_v[pl.ds(k * _CHUNK, 16)]
        for l in range(_CHUNK):
            uid = vec[l]
            col = pl.multiple_of((uid // 128) * 128, 128)
            pltpu.async_copy(tab_t.at[:, pl.ds(col, 128)],
                             ring.at[buf, l], sem)

    def drain_extract(k, buf):
        for l in range(_CHUNK):
            pltpu.make_async_copy(tab_t.at[:, pl.ds(0, 128)],
                                  ring.at[buf, l], sem).wait()
        vec = idx_v[pl.ds(k * _CHUNK, 16)]
        for l in range(_CHUNK):
            uid = vec[l]
            lane = jnp.full((16,), uid % 128, jnp.int32)
            jcol = jnp.full((16,), k * _CHUNK + l, jnp.int32)
            lo = plsc.load_gather(ring.at[buf, l], [iota, lane])
            hi = plsc.load_gather(ring.at[buf, l], [iota + 16, lane])
            plsc.store_scatter(obuf, [iota, jcol], lo)
            plsc.store_scatter(obuf, [iota + 16, jcol], hi)

    fire(0, 0)
    fire(1, 1)

    def pair(p, carry):
        k0 = 2 * p

        @pl.when(k0 + 2 < nchunk)
        def _():
            drain_extract(k0, 0)
            fire(k0 + 2, 0)
            drain_extract(k0 + 1, 1)
            fire(k0 + 3, 1)

        @pl.when(k0 + 2 >= nchunk)
        def _():
            drain_extract(k0, 0)
            drain_extract(k0 + 1, 1)

        return carry

    lax.fori_loop(0, nchunk // 2, pair, 0)


def _sc_gather_body(uids_hbm, iids_hbm, utab_t, itab_t, ut_out, it_out,
                    uidx_v, iidx_v, ring, ubuf, ibuf, sem):
    wid = lax.axis_index("s") * _NC + lax.axis_index("c")
    base = wid * _BPW
    pltpu.sync_copy(uids_hbm.at[pl.ds(base, _BPW)], uidx_v.at[pl.ds(0, _BPW)])
    pltpu.sync_copy(iids_hbm.at[pl.ds(base, _BPW)], iidx_v.at[pl.ds(0, _BPW)])
    _gather_one(utab_t, uidx_v, ring, ubuf, sem)
    _gather_one(itab_t, iidx_v, ring, ibuf, sem)
    pltpu.sync_copy(ubuf, ut_out.at[:, pl.ds(base, _BPW)])
    pltpu.sync_copy(ibuf, it_out.at[:, pl.ds(base, _BPW)])


def _sc_gather(user_ids, item_ids, utab_t, itab_t):
    mesh = plsc.VectorSubcoreMesh(core_axis_name="c", subcore_axis_name="s")
    kfn = pl.kernel(
        _sc_gather_body,
        mesh=mesh,
        out_type=[
            jax.ShapeDtypeStruct((EMB, BATCH), jnp.float32),
            jax.ShapeDtypeStruct((EMB, BATCH), jnp.float32),
        ],
        scratch_types=[
            pltpu.VMEM((_IDXPAD,), jnp.int32),
            pltpu.VMEM((_IDXPAD,), jnp.int32),
            pltpu.VMEM((2, _CHUNK, EMB, 128), jnp.float32),
            pltpu.VMEM((EMB, _BPW), jnp.float32),
            pltpu.VMEM((EMB, _BPW), jnp.float32),
            pltpu.SemaphoreType.DMA,
        ],
        compiler_params=pltpu.CompilerParams(use_tc_tiling_on_sc=True,
                                             needs_layout_passes=False),
    )
    return kfn(user_ids, item_ids, utab_t, itab_t)


def _tc_dense_body(ut_ref, it_ref, w1u_ref, w1i_ref, w1p_ref, b1_ref,
                   w2_ref, b2_ref, pred_ref, score_ref):
    ut = ut_ref[...]
    it = it_ref[...]
    s = jnp.sum(ut, axis=1, keepdims=True)
    pred_ref[...] = jnp.sum(it * s, axis=0, keepdims=True)
    uit = ut * it
    h = (jnp.dot(w1u_ref[...], ut, preferred_element_type=jnp.float32)
         + jnp.dot(w1i_ref[...], it, preferred_element_type=jnp.float32)
         + jnp.dot(w1p_ref[...], uit, preferred_element_type=jnp.float32)
         + b1_ref[...])
    h = jnp.maximum(h, 0.0)
    score_ref[...] = (jnp.sum(h * w2_ref[...], axis=0, keepdims=True)
                      + b2_ref[...])


def _tc_dense(ut, it, W1, b1, W2, b2):
    return pl.pallas_call(
        _tc_dense_body,
        out_shape=[
            jax.ShapeDtypeStruct((1, BATCH), jnp.float32),
            jax.ShapeDtypeStruct((1, BATCH), jnp.float32),
        ],
    )(ut, it, W1[:, :EMB], W1[:, EMB:2 * EMB], W1[:, 2 * EMB:],
      b1.reshape(64, 1), W2.reshape(64, 1), b2.reshape(1, 1))


def kernel(user_ids, item_ids, user_table, item_table, W1, b1, W2, b2):
    uids = user_ids.astype(jnp.int32)
    iids = item_ids.astype(jnp.int32)
    ut, it = _sc_gather(uids, iids, user_table.T, item_table.T)
    pred, score = _tc_dense(ut, it, W1, b1, W2, b2)
    return (pred[0], score[0])


# speedup vs baseline: 12.2542x; 1.0087x over previous
import jax
import jax.numpy as jnp
from jax import lax
from jax.experimental import pallas as pl
from jax.experimental.pallas import tpu as pltpu
from jax.experimental.pallas import tpu_sc as plsc

BATCH = 4096
EMB = 32
_NC = 2
_NS = 16
_NW = _NC * _NS
_BPW = BATCH // _NW
_CHUNK = 8
_IDXPAD = _BPW + 32


def _gather_one(tab_t, idx_v, ring, obuf, sem):
    iota = lax.iota(jnp.int32, 16)
    nchunk = _BPW // _CHUNK

    def fire(k, buf):
        vec = idx_v[pl.ds(k * _CHUNK, 16)]
        for l in range(_CHUNK):
            uid = vec[l]
            col = pl.multiple_of((uid // 128) * 128, 128)
            pltpu.async_copy(tab_t.at[:, pl.ds(col, 128)],
                             ring.at[buf, l], sem)

    def drain_extract(k, buf):
        for l in range(_CHUNK):
            pltpu.make_async_copy(tab_t.at[:, pl.ds(0, 128)],
                                  ring.at[buf, l], sem).wait()
        vec = idx_v[pl.ds(k * _CHUNK, 16)]
        for l in range(_CHUNK):
            uid = vec[l]
            lane = jnp.full((16,), uid % 128, jnp.int32)
            jcol = jnp.full((16,), k * _CHUNK + l, jnp.int32)
            lo = plsc.load_gather(ring.at[buf, l], [iota, lane])
            hi = plsc.load_gather(ring.at[buf, l], [iota + 16, lane])
            plsc.store_scatter(obuf, [iota, jcol], lo)
            plsc.store_scatter(obuf, [iota + 16, jcol], hi)

    fire(0, 0)
    fire(1, 1)
    fire(2, 2)

    def triple(p, carry):
        for q in range(3):
            k = 3 * p + q

            @pl.when(k < nchunk)
            def _(k=k, q=q):
                drain_extract(k, q)

                @pl.when(k + 3 < nchunk)
                def _():
                    fire(k + 3, q)

        return carry

    lax.fori_loop(0, (nchunk + 2) // 3, triple, 0)


def _sc_gather_body(uids_hbm, iids_hbm, utab_t, itab_t, ut_out, it_out,
                    uidx_v, iidx_v, ring, ubuf, ibuf, sem):
    wid = lax.axis_index("s") * _NC + lax.axis_index("c")
    base = wid * _BPW
    pltpu.sync_copy(uids_hbm.at[pl.ds(base, _BPW)], uidx_v.at[pl.ds(0, _BPW)])
    pltpu.sync_copy(iids_hbm.at[pl.ds(base, _BPW)], iidx_v.at[pl.ds(0, _BPW)])
    _gather_one(utab_t, uidx_v, ring, ubuf, sem)
    _gather_one(itab_t, iidx_v, ring, ibuf, sem)
    pltpu.sync_copy(ubuf, ut_out.at[:, pl.ds(base, _BPW)])
    pltpu.sync_copy(ibuf, it_out.at[:, pl.ds(base, _BPW)])


def _sc_gather(user_ids, item_ids, utab_t, itab_t):
    mesh = plsc.VectorSubcoreMesh(core_axis_name="c", subcore_axis_name="s")
    kfn = pl.kernel(
        _sc_gather_body,
        mesh=mesh,
        out_type=[
            jax.ShapeDtypeStruct((EMB, BATCH), jnp.float32),
            jax.ShapeDtypeStruct((EMB, BATCH), jnp.float32),
        ],
        scratch_types=[
            pltpu.VMEM((_IDXPAD,), jnp.int32),
            pltpu.VMEM((_IDXPAD,), jnp.int32),
            pltpu.VMEM((3, _CHUNK, EMB, 128), jnp.float32),
            pltpu.VMEM((EMB, _BPW), jnp.float32),
            pltpu.VMEM((EMB, _BPW), jnp.float32),
            pltpu.SemaphoreType.DMA,
        ],
        compiler_params=pltpu.CompilerParams(use_tc_tiling_on_sc=True,
                                             needs_layout_passes=False),
    )
    return kfn(user_ids, item_ids, utab_t, itab_t)


def _tc_dense_body(ut_ref, it_ref, w1u_ref, w1i_ref, w1p_ref, b1_ref,
                   w2_ref, b2_ref, pred_ref, score_ref):
    ut = ut_ref[...]
    it = it_ref[...]
    s = jnp.sum(ut, axis=1, keepdims=True)
    pred_ref[...] = jnp.sum(it * s, axis=0, keepdims=True)
    uit = ut * it
    h = (jnp.dot(w1u_ref[...], ut, preferred_element_type=jnp.float32)
         + jnp.dot(w1i_ref[...], it, preferred_element_type=jnp.float32)
         + jnp.dot(w1p_ref[...], uit, preferred_element_type=jnp.float32)
         + b1_ref[...])
    h = jnp.maximum(h, 0.0)
    score_ref[...] = (jnp.sum(h * w2_ref[...], axis=0, keepdims=True)
                      + b2_ref[...])


def _tc_dense(ut, it, W1, b1, W2, b2):
    return pl.pallas_call(
        _tc_dense_body,
        out_shape=[
            jax.ShapeDtypeStruct((1, BATCH), jnp.float32),
            jax.ShapeDtypeStruct((1, BATCH), jnp.float32),
        ],
    )(ut, it, W1[:, :EMB], W1[:, EMB:2 * EMB], W1[:, 2 * EMB:],
      b1.reshape(64, 1), W2.reshape(64, 1), b2.reshape(1, 1))


def kernel(user_ids, item_ids, user_table, item_table, W1, b1, W2, b2):
    uids = user_ids.astype(jnp.int32)
    iids = item_ids.astype(jnp.int32)
    ut, it = _sc_gather(uids, iids, user_table.T, item_table.T)
    pred, score = _tc_dense(ut, it, W1, b1, W2, b2)
    return (pred[0], score[0])
